# baseline (speedup 1.0000x reference)
import functools

import jax
import jax.numpy as jnp
from jax import lax
from jax.experimental import pallas as pl
from jax.experimental.pallas import tpu as pltpu
from jax.experimental.pallas import tpu_sc as plsc

_M = 1024
_NW = 32
_LANES = 16


def _make_sc_gather(num_rows, n_cols, rows_per_worker, chunk):
    mesh = plsc.VectorSubcoreMesh(core_axis_name="c", subcore_axis_name="s")

    @functools.partial(
        pl.kernel,
        mesh=mesh,
        out_type=jax.ShapeDtypeStruct((num_rows * _M,), jnp.float32),
        scratch_types=[
            pltpu.VMEM((_M,), jnp.int32),
            pltpu.VMEM((chunk * _M,), jnp.float32),
            pltpu.VMEM((chunk * _M,), jnp.float32),
        ],
        compiler_params=pltpu.CompilerParams(needs_layout_passes=False),
    )
    def k(x_hbm, idx_hbm, out_hbm, idx_v, in_v, out_v):
        wid = lax.axis_index("s") * 2 + lax.axis_index("c")
        base = wid * rows_per_worker
        pltpu.sync_copy(idx_hbm, idx_v)

        def chunk_body(ci, carry):
            row0 = base + ci * chunk
            for r in range(chunk):
                pltpu.sync_copy(
                    x_hbm.at[pl.ds((row0 + r) * n_cols, _M)],
                    in_v.at[pl.ds(r * _M, _M)],
                )

            def j_body(j, c2):
                col = idx_v[pl.ds(j * _LANES, _LANES)]
                for r in range(chunk):
                    flat = col + jnp.full((_LANES,), r * _M, jnp.int32)
                    out_v[pl.ds(r * _M + j * _LANES, _LANES)] = (
                        plsc.load_gather(in_v, [flat])
                    )
                return c2

            lax.fori_loop(0, _M // _LANES, j_body, 0)
            pltpu.sync_copy(out_v, out_hbm.at[pl.ds(row0 * _M, chunk * _M)])
            return carry

        lax.fori_loop(0, rows_per_worker // chunk, chunk_body, 0)

    return k


def kernel(x):
    b, c, n = x.shape
    num_rows = b * c
    idx = jax.random.randint(
        jax.random.key(1), (1, _M), 0, _M, dtype=jnp.int32
    )[0]
    rows_per_worker = num_rows // _NW
    chunk = min(32, rows_per_worker)
    gather = _make_sc_gather(num_rows, n, rows_per_worker, chunk)
    y = gather(x.reshape(num_rows * n), idx)
    return y.reshape(b, c, _M)


# speedup vs baseline: 2.6352x; 2.6352x over previous
import functools

import jax
import jax.numpy as jnp
from jax import lax
from jax.experimental import pallas as pl
from jax.experimental.pallas import tpu as pltpu
from jax.experimental.pallas import tpu_sc as plsc

_M = 1024
_NW = 32
_LANES = 16
_CH = 16


def _make_sc_gather(num_rows, n_cols, rows_per_worker):
    mesh = plsc.VectorSubcoreMesh(core_axis_name="c", subcore_axis_name="s")
    nch = rows_per_worker // _CH

    @functools.partial(
        pl.kernel,
        mesh=mesh,
        out_type=jax.ShapeDtypeStruct((num_rows, _M), jnp.float32),
        scratch_types=[
            pltpu.VMEM((_M,), jnp.int32),
            pltpu.VMEM((_CH, _M), jnp.float32),
            pltpu.VMEM((_CH, _M), jnp.float32),
            pltpu.VMEM((_CH, _M), jnp.float32),
            pltpu.VMEM((_CH, _M), jnp.float32),
            pltpu.SemaphoreType.DMA,
            pltpu.SemaphoreType.DMA,
            pltpu.SemaphoreType.DMA,
            pltpu.SemaphoreType.DMA,
            pltpu.SemaphoreType.DMA,
        ],
        compiler_params=pltpu.CompilerParams(needs_layout_passes=False),
    )
    def k(x_hbm, idx_hbm, out_hbm, idx_v, in0, in1, out0, out1,
          s_in0, s_in1, s_out0, s_out1, s_idx):
        wid = lax.axis_index("s") * 2 + lax.axis_index("c")
        base = wid * rows_per_worker
        ins, outs = (in0, in1), (out0, out1)
        s_ins, s_outs = (s_in0, s_in1), (s_out0, s_out1)

        def in_copy(ci):
            row0 = base + ci * _CH
            return pltpu.make_async_copy(
                x_hbm.at[pl.ds(row0, _CH), pl.ds(0, _M)],
                ins[ci % 2], s_ins[ci % 2])

        def out_copy(ci):
            row0 = base + ci * _CH
            return pltpu.make_async_copy(
                outs[ci % 2], out_hbm.at[pl.ds(row0, _CH), :],
                s_outs[ci % 2])

        idx_cp = pltpu.make_async_copy(idx_hbm, idx_v, s_idx)
        idx_cp.start()
        in_copy(0).start()
        in_copy(1).start()
        idx_cp.wait()

        for ci in range(nch):
            in_copy(ci).wait()
            if ci >= 2:
                out_copy(ci - 2).wait()
            src, dst = ins[ci % 2], outs[ci % 2]

            def j_body(j, c2, src=src, dst=dst):
                col = idx_v[pl.ds(j * _LANES, _LANES)]
                for r in range(_CH):
                    rid = jnp.full((_LANES,), r, jnp.int32)
                    dst[r, pl.ds(j * _LANES, _LANES)] = plsc.load_gather(
                        src, [rid, col])
                return c2

            lax.fori_loop(0, _M // _LANES, j_body, 0)
            out_copy(ci).start()
            if ci + 2 < nch:
                in_copy(ci + 2).start()

        out_copy(nch - 2).wait()
        out_copy(nch - 1).wait()

    return k


def kernel(x):
    b, c, n = x.shape
    num_rows = b * c
    idx = jax.random.randint(
        jax.random.key(1), (1, _M), 0, _M, dtype=jnp.int32
    )[0]
    rows_per_worker = num_rows // _NW
    gather = _make_sc_gather(num_rows, n, rows_per_worker)
    y = gather(x.reshape(num_rows, n), idx)
    return y.reshape(b, c, _M)


# speedup vs baseline: 3.4233x; 1.2991x over previous
import functools

import jax
import jax.numpy as jnp
from jax import lax
from jax.experimental import pallas as pl
from jax.experimental.pallas import tpu as pltpu
from jax.experimental.pallas import tpu_sc as plsc

_M = 1024
_NW = 32
_LANES = 16
_CH = 16


def _make_sc_gather(num_rows, n_cols, rows_per_worker):
    mesh = plsc.VectorSubcoreMesh(core_axis_name="c", subcore_axis_name="s")
    nch = rows_per_worker // _CH

    @functools.partial(
        pl.kernel,
        mesh=mesh,
        out_type=jax.ShapeDtypeStruct((num_rows, _M), jnp.float32),
        scratch_types=[
            pltpu.VMEM((_M,), jnp.int32),
            pltpu.VMEM((_CH, _M), jnp.float32),
            pltpu.VMEM((_CH, _M), jnp.float32),
            pltpu.VMEM((_CH, _M), jnp.float32),
            pltpu.VMEM((_CH, _M), jnp.float32),
            pltpu.SemaphoreType.DMA,
            pltpu.SemaphoreType.DMA,
            pltpu.SemaphoreType.DMA,
            pltpu.SemaphoreType.DMA,
            pltpu.SemaphoreType.DMA,
        ],
        compiler_params=pltpu.CompilerParams(needs_layout_passes=False),
    )
    def k(x_hbm, idx_hbm, out_hbm, idx_v, in0, in1, out0, out1,
          s_in0, s_in1, s_out0, s_out1, s_idx):
        wid = lax.axis_index("s") * 2 + lax.axis_index("c")
        base = wid * rows_per_worker
        ins, outs = (in0, in1), (out0, out1)
        s_ins, s_outs = (s_in0, s_in1), (s_out0, s_out1)

        def in_copy(ci):
            row0 = base + ci * _CH
            return pltpu.make_async_copy(
                x_hbm.at[pl.ds(row0, _CH), pl.ds(0, _M)],
                ins[ci % 2], s_ins[ci % 2])

        def out_copy(ci):
            row0 = base + ci * _CH
            return pltpu.make_async_copy(
                outs[ci % 2], out_hbm.at[pl.ds(row0, _CH), :],
                s_outs[ci % 2])

        idx_cp = pltpu.make_async_copy(idx_hbm, idx_v, s_idx)
        idx_cp.start()
        in_copy(0).start()
        in_copy(1).start()
        idx_cp.wait()

        for ci in range(nch):
            in_copy(ci).wait()
            if ci >= 2:
                out_copy(ci - 2).wait()
            src, dst = ins[ci % 2], outs[ci % 2]

            def j_body(j, c2, src=src, dst=dst):
                col = idx_v[pl.ds(j * _LANES, _LANES)]
                vals = [
                    plsc.load_gather(
                        src, [jnp.full((_LANES,), r, jnp.int32), col])
                    for r in range(_CH)
                ]
                for r in range(_CH):
                    dst[r, pl.ds(j * _LANES, _LANES)] = vals[r]
                return c2

            lax.fori_loop(0, _M // _LANES, j_body, 0)
            out_copy(ci).start()
            if ci + 2 < nch:
                in_copy(ci + 2).start()

        out_copy(nch - 2).wait()
        out_copy(nch - 1).wait()

    return k


def kernel(x):
    b, c, n = x.shape
    num_rows = b * c
    idx = jax.random.randint(
        jax.random.key(1), (1, _M), 0, _M, dtype=jnp.int32
    )[0]
    rows_per_worker = num_rows // _NW
    gather = _make_sc_gather(num_rows, n, rows_per_worker)
    y = gather(x.reshape(num_rows, n), idx)
    return y.reshape(b, c, _M)


# speedup vs baseline: 3.6704x; 1.0722x over previous
import functools

import jax
import jax.numpy as jnp
import numpy as np
from jax import lax
from jax.experimental import pallas as pl
from jax.experimental.pallas import tpu as pltpu
from jax.experimental.pallas import tpu_sc as plsc

_M = 1024
_NW = 32
_LANES = 16
_CH = 16


def _rotl32(x, d):
    return ((x << np.uint32(d)) | (x >> np.uint32(32 - d))).astype(np.uint32)


def _threefry2x32(k0, k1, x0, x1):
    rotations = ((13, 15, 26, 6), (17, 29, 16, 24))
    ks = (np.uint32(k0), np.uint32(k1),
          np.uint32(k0) ^ np.uint32(k1) ^ np.uint32(0x1BD11BDA))
    a = (x0 + ks[0]).astype(np.uint32)
    b = (x1 + ks[1]).astype(np.uint32)
    for i in range(5):
        for r in rotations[i % 2]:
            a = (a + b).astype(np.uint32)
            b = a ^ _rotl32(b, r)
        a = (a + ks[(i + 1) % 3]).astype(np.uint32)
        b = (b + ks[(i + 2) % 3] + np.uint32(i + 1)).astype(np.uint32)
    return a, b


def _sampler_indices():
    one = np.array([1], np.uint32)
    zero = np.array([0], np.uint32)
    sk_a, sk_b = _threefry2x32(np.uint32(0), np.uint32(1), zero, one)
    counts = np.arange(_M, dtype=np.uint32)
    a, b = _threefry2x32(sk_a[0], sk_b[0],
                         np.zeros(_M, np.uint32), counts)
    return ((a ^ b) % np.uint32(_M)).astype(np.int32)


_IDX = _sampler_indices()


def _make_sc_gather(num_rows, n_cols, rows_per_worker):
    mesh = plsc.VectorSubcoreMesh(core_axis_name="c", subcore_axis_name="s")
    nch = rows_per_worker // _CH

    @functools.partial(
        pl.kernel,
        mesh=mesh,
        out_type=jax.ShapeDtypeStruct((num_rows, _M), jnp.float32),
        scratch_types=[
            pltpu.VMEM((_M,), jnp.int32),
            pltpu.VMEM((_CH, _M), jnp.float32),
            pltpu.VMEM((_CH, _M), jnp.float32),
            pltpu.VMEM((_CH, _M), jnp.float32),
            pltpu.VMEM((_CH, _M), jnp.float32),
            pltpu.SemaphoreType.DMA,
            pltpu.SemaphoreType.DMA,
            pltpu.SemaphoreType.DMA,
            pltpu.SemaphoreType.DMA,
            pltpu.SemaphoreType.DMA,
        ],
        compiler_params=pltpu.CompilerParams(needs_layout_passes=False),
    )
    def k(x_hbm, idx_hbm, out_hbm, idx_v, in0, in1, out0, out1,
          s_in0, s_in1, s_out0, s_out1, s_idx):
        wid = lax.axis_index("s") * 2 + lax.axis_index("c")
        base = wid * rows_per_worker
        ins, outs = (in0, in1), (out0, out1)
        s_ins, s_outs = (s_in0, s_in1), (s_out0, s_out1)

        def in_copy(ci):
            row0 = base + ci * _CH
            return pltpu.make_async_copy(
                x_hbm.at[pl.ds(row0, _CH), pl.ds(0, _M)],
                ins[ci % 2], s_ins[ci % 2])

        def out_copy(ci):
            row0 = base + ci * _CH
            return pltpu.make_async_copy(
                outs[ci % 2], out_hbm.at[pl.ds(row0, _CH), :],
                s_outs[ci % 2])

        idx_cp = pltpu.make_async_copy(idx_hbm, idx_v, s_idx)
        idx_cp.start()
        in_copy(0).start()
        in_copy(1).start()
        idx_cp.wait()

        for ci in range(nch):
            in_copy(ci).wait()
            if ci >= 2:
                out_copy(ci - 2).wait()
            src, dst = ins[ci % 2], outs[ci % 2]

            def j_body(j, c2, src=src, dst=dst):
                col = idx_v[pl.ds(j * _LANES, _LANES)]
                vals = [
                    plsc.load_gather(
                        src, [jnp.full((_LANES,), r, jnp.int32), col])
                    for r in range(_CH)
                ]
                for r in range(_CH):
                    dst[r, pl.ds(j * _LANES, _LANES)] = vals[r]
                return c2

            lax.fori_loop(0, _M // _LANES, j_body, 0)
            out_copy(ci).start()
            if ci + 2 < nch:
                in_copy(ci + 2).start()

        out_copy(nch - 2).wait()
        out_copy(nch - 1).wait()

    return k


def kernel(x):
    b, c, n = x.shape
    num_rows = b * c
    idx = jnp.asarray(_IDX)
    rows_per_worker = num_rows // _NW
    gather = _make_sc_gather(num_rows, n, rows_per_worker)
    y = gather(x.reshape(num_rows, n), idx)
    return y.reshape(b, c, _M)
